# baseline (speedup 1.0000x reference)
import functools

import jax
import jax.numpy as jnp
from jax import lax
from jax.experimental import pallas as pl
from jax.experimental.pallas import tpu as pltpu
from jax.experimental.pallas import tpu_sc as plsc

_NC = 2
_NS = 16
_NW = _NC * _NS
_CH = 8
_U = 4
_K = 3


def _sc_add(x2, pos2, B, S, D):
    rows_per_w = S // _NW
    chunks = rows_per_w // _CH
    VECS = (_CH * D) // 16
    minor_vecs = D // 16

    mesh = plsc.VectorSubcoreMesh(core_axis_name="c", subcore_axis_name="s")

    @functools.partial(
        pl.kernel,
        mesh=mesh,
        out_type=jax.ShapeDtypeStruct((B * S, D), jnp.float32),
        scratch_types=(
            [pltpu.VMEM((_CH, D), jnp.float32) for _ in range(2 + _K * B)]
            + [pltpu.SemaphoreType.DMA for _ in range(2 + 2 * _K)]
        ),
    )
    def k(x_hbm, pos_hbm, out_hbm, *scratch):
        pos_bufs = scratch[0:2]
        x_slots = tuple(scratch[2 + i * B:2 + (i + 1) * B] for i in range(_K))
        pos_sems = scratch[2 + _K * B:4 + _K * B]
        in_sems = scratch[4 + _K * B:4 + _K * B + _K]
        out_sems = scratch[4 + _K * B + _K:4 + _K * B + 2 * _K]

        wid = lax.axis_index("s") * _NC + lax.axis_index("c")
        base_row = wid * rows_per_w

        def pos_slice(c):
            return pos_hbm.at[pl.ds(pl.multiple_of(base_row + c * _CH, _CH), _CH), :]

        def x_slice(hbm, c, b):
            row = pl.multiple_of(b * S + base_row + c * _CH, _CH)
            return hbm.at[pl.ds(row, _CH), :]

        def issue_in(c):
            slot = c % _K
            return [pltpu.async_copy(x_slice(x_hbm, c, b), x_slots[slot][b],
                                     in_sems[slot])
                    for b in range(B)]

        pos_desc = {0: pltpu.async_copy(pos_slice(0), pos_bufs[0], pos_sems[0])}
        in_desc = {0: issue_in(0)}
        out_desc = {}
        out_waited = set()

        for c in range(chunks):
            slot = c % _K
            if c + 1 < chunks:
                prev = c + 1 - _K
                if prev >= 0:
                    for d in out_desc[prev]:
                        d.wait()
                    out_waited.add(prev)
                pos_desc[c + 1] = pltpu.async_copy(
                    pos_slice(c + 1), pos_bufs[(c + 1) % 2],
                    pos_sems[(c + 1) % 2])
                in_desc[c + 1] = issue_in(c + 1)
            pos_desc[c].wait()
            for d in in_desc[c]:
                d.wait()

            bufs = x_slots[slot]
            pv = pos_bufs[c % 2]

            @plsc.parallel_loop(0, VECS, step=1, unroll=_U)
            def add_u(i, bufs=bufs, pv=pv):
                r = i // minor_vecs
                sl = pl.ds((i % minor_vecs) * 16, 16)
                pvv = pv[r, sl]
                for b in range(B):
                    bufs[b][r, sl] = bufs[b][r, sl] + pvv

            out_desc[c] = [pltpu.async_copy(bufs[b], x_slice(out_hbm, c, b),
                                            out_sems[slot])
                           for b in range(B)]

        for c in range(chunks):
            if c not in out_waited:
                for d in out_desc[c]:
                    d.wait()

    return k(x2, pos2)


def kernel(x, pos_table):
    B, S, D = x.shape
    out2 = _sc_add(x.reshape(B * S, D), pos_table[:S], B, S, D)
    return out2.reshape(B, S, D)


# speedup vs baseline: 1.0017x; 1.0017x over previous
import functools

import jax
import jax.numpy as jnp
from jax import lax
from jax.experimental import pallas as pl
from jax.experimental.pallas import tpu as pltpu
from jax.experimental.pallas import tpu_sc as plsc

_NC = 2
_NS = 16
_NW = _NC * _NS
_CH = 8
_U = 4
_K = 3


def _sc_add(x2, pos2, B, S, D):
    rows_per_w = S // _NW
    chunks = rows_per_w // _CH
    VECS = (_CH * D) // 16
    minor_vecs = D // 16

    mesh = plsc.VectorSubcoreMesh(core_axis_name="c", subcore_axis_name="s")

    @functools.partial(
        pl.kernel,
        mesh=mesh,
        out_type=jax.ShapeDtypeStruct((B * S, D), jnp.float32),
        compiler_params=pltpu.CompilerParams(
            disable_bounds_checks=True,
            disable_semaphore_checks=True,
        ),
        scratch_types=(
            [pltpu.VMEM((_CH, D), jnp.float32) for _ in range(2 + _K * B)]
            + [pltpu.SemaphoreType.DMA for _ in range(2 + 2 * _K)]
        ),
    )
    def k(x_hbm, pos_hbm, out_hbm, *scratch):
        pos_bufs = scratch[0:2]
        x_slots = tuple(scratch[2 + i * B:2 + (i + 1) * B] for i in range(_K))
        pos_sems = scratch[2 + _K * B:4 + _K * B]
        in_sems = scratch[4 + _K * B:4 + _K * B + _K]
        out_sems = scratch[4 + _K * B + _K:4 + _K * B + 2 * _K]

        wid = lax.axis_index("s") * _NC + lax.axis_index("c")
        base_row = wid * rows_per_w

        def pos_slice(c):
            return pos_hbm.at[pl.ds(pl.multiple_of(base_row + c * _CH, _CH), _CH), :]

        def x_slice(hbm, c, b):
            row = pl.multiple_of(b * S + base_row + c * _CH, _CH)
            return hbm.at[pl.ds(row, _CH), :]

        def issue_in(c):
            slot = c % _K
            return [pltpu.async_copy(x_slice(x_hbm, c, b), x_slots[slot][b],
                                     in_sems[slot])
                    for b in range(B)]

        pos_desc = {0: pltpu.async_copy(pos_slice(0), pos_bufs[0], pos_sems[0])}
        in_desc = {0: issue_in(0)}
        out_desc = {}
        out_waited = set()

        for c in range(chunks):
            slot = c % _K
            if c + 1 < chunks:
                prev = c + 1 - _K
                if prev >= 0:
                    for d in out_desc[prev]:
                        d.wait()
                    out_waited.add(prev)
                pos_desc[c + 1] = pltpu.async_copy(
                    pos_slice(c + 1), pos_bufs[(c + 1) % 2],
                    pos_sems[(c + 1) % 2])
                in_desc[c + 1] = issue_in(c + 1)
            pos_desc[c].wait()
            for d in in_desc[c]:
                d.wait()

            bufs = x_slots[slot]
            pv = pos_bufs[c % 2]

            @plsc.parallel_loop(0, VECS, step=1, unroll=_U)
            def add_u(i, bufs=bufs, pv=pv):
                r = i // minor_vecs
                sl = pl.ds((i % minor_vecs) * 16, 16)
                pvv = pv[r, sl]
                for b in range(B):
                    bufs[b][r, sl] = bufs[b][r, sl] + pvv

            out_desc[c] = [pltpu.async_copy(bufs[b], x_slice(out_hbm, c, b),
                                            out_sems[slot])
                           for b in range(B)]

        for c in range(chunks):
            if c not in out_waited:
                for d in out_desc[c]:
                    d.wait()

    return k(x2, pos2)


def kernel(x, pos_table):
    B, S, D = x.shape
    out2 = _sc_add(x.reshape(B * S, D), pos_table[:S], B, S, D)
    return out2.reshape(B, S, D)


# speedup vs baseline: 1.0044x; 1.0028x over previous
import functools

import jax
import jax.numpy as jnp
from jax import lax
from jax.experimental import pallas as pl
from jax.experimental.pallas import tpu as pltpu
from jax.experimental.pallas import tpu_sc as plsc

_NC = 2
_NS = 16
_NW = _NC * _NS
_CH = 8
_U = 4
_K = 3


def _sc_add(x2, pos2, B, S, D):
    rows_per_w = S // _NW
    chunks = rows_per_w // _CH
    VECS = (_CH * D) // 16
    minor_vecs = D // 16

    mesh = plsc.VectorSubcoreMesh(core_axis_name="c", subcore_axis_name="s")

    @functools.partial(
        pl.kernel,
        mesh=mesh,
        out_type=jax.ShapeDtypeStruct((B * S, D), jnp.float32),
        compiler_params=pltpu.CompilerParams(
            disable_bounds_checks=True,
            disable_semaphore_checks=True,
            skip_device_barrier=True,
        ),
        scratch_types=(
            [pltpu.VMEM((_CH, D), jnp.float32) for _ in range(2 + _K * B)]
            + [pltpu.SemaphoreType.DMA for _ in range(2 + 2 * _K)]
        ),
    )
    def k(x_hbm, pos_hbm, out_hbm, *scratch):
        pos_bufs = scratch[0:2]
        x_slots = tuple(scratch[2 + i * B:2 + (i + 1) * B] for i in range(_K))
        pos_sems = scratch[2 + _K * B:4 + _K * B]
        in_sems = scratch[4 + _K * B:4 + _K * B + _K]
        out_sems = scratch[4 + _K * B + _K:4 + _K * B + 2 * _K]

        wid = lax.axis_index("s") * _NC + lax.axis_index("c")
        base_row = wid * rows_per_w

        def pos_slice(c):
            return pos_hbm.at[pl.ds(pl.multiple_of(base_row + c * _CH, _CH), _CH), :]

        def x_slice(hbm, c, b):
            row = pl.multiple_of(b * S + base_row + c * _CH, _CH)
            return hbm.at[pl.ds(row, _CH), :]

        def issue_in(c):
            slot = c % _K
            return [pltpu.async_copy(x_slice(x_hbm, c, b), x_slots[slot][b],
                                     in_sems[slot])
                    for b in range(B)]

        pos_desc = {0: pltpu.async_copy(pos_slice(0), pos_bufs[0], pos_sems[0])}
        in_desc = {0: issue_in(0)}
        out_desc = {}
        out_waited = set()

        for c in range(chunks):
            slot = c % _K
            if c + 1 < chunks:
                prev = c + 1 - _K
                if prev >= 0:
                    for d in out_desc[prev]:
                        d.wait()
                    out_waited.add(prev)
                pos_desc[c + 1] = pltpu.async_copy(
                    pos_slice(c + 1), pos_bufs[(c + 1) % 2],
                    pos_sems[(c + 1) % 2])
                in_desc[c + 1] = issue_in(c + 1)
            pos_desc[c].wait()
            for d in in_desc[c]:
                d.wait()

            bufs = x_slots[slot]
            pv = pos_bufs[c % 2]

            @plsc.parallel_loop(0, VECS, step=1, unroll=_U)
            def add_u(i, bufs=bufs, pv=pv):
                r = i // minor_vecs
                sl = pl.ds((i % minor_vecs) * 16, 16)
                pvv = pv[r, sl]
                for b in range(B):
                    bufs[b][r, sl] = bufs[b][r, sl] + pvv

            out_desc[c] = [pltpu.async_copy(bufs[b], x_slice(out_hbm, c, b),
                                            out_sems[slot])
                           for b in range(B)]

        for c in range(chunks):
            if c not in out_waited:
                for d in out_desc[c]:
                    d.wait()

    return k(x2, pos2)


def kernel(x, pos_table):
    B, S, D = x.shape
    out2 = _sc_add(x.reshape(B * S, D), pos_table[:S], B, S, D)
    return out2.reshape(B, S, D)


# speedup vs baseline: 1.0395x; 1.0349x over previous
import functools

import jax
import jax.numpy as jnp
from jax import lax
from jax.experimental import pallas as pl
from jax.experimental.pallas import tpu as pltpu
from jax.experimental.pallas import tpu_sc as plsc

_NC = 2
_NS = 16
_NW = _NC * _NS
_CH = 8
_U = 4
_K = 3


def _sc_add(x2, pos2, B, S, D):
    rows_per_w = S // _NW
    chunks = rows_per_w // _CH
    VECS = (_CH * D) // 16
    minor_vecs = D // 16

    mesh = plsc.VectorSubcoreMesh(core_axis_name="c", subcore_axis_name="s")

    @functools.partial(
        pl.kernel,
        mesh=mesh,
        out_type=jax.ShapeDtypeStruct((B * S, D), jnp.float32),
        compiler_params=pltpu.CompilerParams(
            disable_bounds_checks=True,
            disable_semaphore_checks=True,
        ),
        scratch_types=[
            pltpu.VMEM((2, _CH, D), jnp.float32),
            pltpu.VMEM((_K * B, _CH, D), jnp.float32),
            pltpu.SemaphoreType.DMA((2,)),
            pltpu.SemaphoreType.DMA((_K,)),
            pltpu.SemaphoreType.DMA((_K,)),
        ],
    )
    def k(x_hbm, pos_hbm, out_hbm, pos_buf, x_buf, pos_sem, in_sem, out_sem):
        wid = lax.axis_index("s") * _NC + lax.axis_index("c")
        base_row = wid * rows_per_w

        def pos_slice(c):
            return pos_hbm.at[pl.ds(pl.multiple_of(base_row + c * _CH, _CH), _CH), :]

        def x_slice(hbm, c, b):
            row = pl.multiple_of(b * S + base_row + c * _CH, _CH)
            return hbm.at[pl.ds(row, _CH), :]

        def issue_pos(c, ps):
            pltpu.async_copy(pos_slice(c), pos_buf.at[ps], pos_sem.at[ps])

        def wait_pos(c, ps):
            pltpu.make_async_copy(pos_slice(c), pos_buf.at[ps],
                                  pos_sem.at[ps]).wait()

        def issue_in(c, slot):
            for b in range(B):
                pltpu.async_copy(x_slice(x_hbm, c, b),
                                 x_buf.at[slot * B + b], in_sem.at[slot])

        def wait_in(c, slot):
            for b in range(B):
                pltpu.make_async_copy(x_slice(x_hbm, c, b),
                                      x_buf.at[slot * B + b],
                                      in_sem.at[slot]).wait()

        def issue_out(c, slot):
            for b in range(B):
                pltpu.async_copy(x_buf.at[slot * B + b],
                                 x_slice(out_hbm, c, b), out_sem.at[slot])

        def wait_out(c, slot):
            for b in range(B):
                pltpu.make_async_copy(x_buf.at[slot * B + b],
                                      x_slice(out_hbm, c, b),
                                      out_sem.at[slot]).wait()

        issue_pos(0, 0)
        issue_in(0, 0)

        def chunk_body(c, carry):
            slot = lax.rem(c, _K)
            ps = lax.rem(c, 2)

            @pl.when(c + 1 < chunks)
            def _prefetch():
                nslot = lax.rem(c + 1, _K)
                nps = lax.rem(c + 1, 2)

                @pl.when(c >= _K - 1)
                def _recycle():
                    wait_out(c + 1 - _K, nslot)

                issue_pos(c + 1, nps)
                issue_in(c + 1, nslot)

            wait_pos(c, ps)
            wait_in(c, slot)

            @plsc.parallel_loop(0, VECS, step=1, unroll=_U)
            def add_u(i):
                r = i // minor_vecs
                sl = pl.ds((i % minor_vecs) * 16, 16)
                pvv = pos_buf[ps, r, sl]
                for b in range(B):
                    x_buf[slot * B + b, r, sl] = x_buf[slot * B + b, r, sl] + pvv

            issue_out(c, slot)
            return carry

        lax.fori_loop(0, chunks, chunk_body, 0)

        for c in range(max(chunks - _K, 0), chunks):
            wait_out(c, c % _K)

    return k(x2, pos2)


def kernel(x, pos_table):
    B, S, D = x.shape
    out2 = _sc_add(x.reshape(B * S, D), pos_table[:S], B, S, D)
    return out2.reshape(B, S, D)


# speedup vs baseline: 1.0445x; 1.0048x over previous
import functools

import jax
import jax.numpy as jnp
from jax import lax
from jax.experimental import pallas as pl
from jax.experimental.pallas import tpu as pltpu
from jax.experimental.pallas import tpu_sc as plsc

_NC = 2
_NS = 16
_NW = _NC * _NS
_CH = 8
_U = 8
_K = 3


def _sc_add(x2, pos2, B, S, D):
    rows_per_w = S // _NW
    chunks = rows_per_w // _CH
    VECS = (_CH * D) // 16
    minor_vecs = D // 16

    mesh = plsc.VectorSubcoreMesh(core_axis_name="c", subcore_axis_name="s")

    @functools.partial(
        pl.kernel,
        mesh=mesh,
        out_type=jax.ShapeDtypeStruct((B * S, D), jnp.float32),
        compiler_params=pltpu.CompilerParams(
            disable_bounds_checks=True,
            disable_semaphore_checks=True,
        ),
        scratch_types=[
            pltpu.VMEM((2, _CH, D), jnp.float32),
            pltpu.VMEM((_K * B, _CH, D), jnp.float32),
            pltpu.SemaphoreType.DMA((2,)),
            pltpu.SemaphoreType.DMA((_K,)),
            pltpu.SemaphoreType.DMA((_K,)),
        ],
    )
    def k(x_hbm, pos_hbm, out_hbm, pos_buf, x_buf, pos_sem, in_sem, out_sem):
        wid = lax.axis_index("s") * _NC + lax.axis_index("c")
        base_row = wid * rows_per_w

        def pos_slice(c):
            return pos_hbm.at[pl.ds(pl.multiple_of(base_row + c * _CH, _CH), _CH), :]

        def x_slice(hbm, c, b):
            row = pl.multiple_of(b * S + base_row + c * _CH, _CH)
            return hbm.at[pl.ds(row, _CH), :]

        def issue_pos(c, ps):
            pltpu.async_copy(pos_slice(c), pos_buf.at[ps], pos_sem.at[ps])

        def wait_pos(c, ps):
            pltpu.make_async_copy(pos_slice(c), pos_buf.at[ps],
                                  pos_sem.at[ps]).wait()

        def issue_in(c, slot):
            for b in range(B):
                pltpu.async_copy(x_slice(x_hbm, c, b),
                                 x_buf.at[slot * B + b], in_sem.at[slot])

        def wait_in(c, slot):
            for b in range(B):
                pltpu.make_async_copy(x_slice(x_hbm, c, b),
                                      x_buf.at[slot * B + b],
                                      in_sem.at[slot]).wait()

        def issue_out(c, slot):
            for b in range(B):
                pltpu.async_copy(x_buf.at[slot * B + b],
                                 x_slice(out_hbm, c, b), out_sem.at[slot])

        def wait_out(c, slot):
            for b in range(B):
                pltpu.make_async_copy(x_buf.at[slot * B + b],
                                      x_slice(out_hbm, c, b),
                                      out_sem.at[slot]).wait()

        issue_pos(0, 0)
        issue_in(0, 0)

        def chunk_body(c, carry):
            slot = lax.rem(c, _K)
            ps = lax.rem(c, 2)

            @pl.when(c + 1 < chunks)
            def _prefetch():
                nslot = lax.rem(c + 1, _K)
                nps = lax.rem(c + 1, 2)

                @pl.when(c >= _K - 1)
                def _recycle():
                    wait_out(c + 1 - _K, nslot)

                issue_pos(c + 1, nps)
                issue_in(c + 1, nslot)

            wait_pos(c, ps)
            wait_in(c, slot)

            @plsc.parallel_loop(0, VECS, step=1, unroll=_U)
            def add_u(i):
                r = i // minor_vecs
                sl = pl.ds((i % minor_vecs) * 16, 16)
                pvv = pos_buf[ps, r, sl]
                for b in range(B):
                    x_buf[slot * B + b, r, sl] = x_buf[slot * B + b, r, sl] + pvv

            issue_out(c, slot)
            return carry

        lax.fori_loop(0, chunks, chunk_body, 0)

        for c in range(max(chunks - _K, 0), chunks):
            wait_out(c, c % _K)

    return k(x2, pos2)


def kernel(x, pos_table):
    B, S, D = x.shape
    out2 = _sc_add(x.reshape(B * S, D), pos_table[:S], B, S, D)
    return out2.reshape(B, S, D)


# speedup vs baseline: 1.0523x; 1.0075x over previous
import functools

import jax
import jax.numpy as jnp
from jax import lax
from jax.experimental import pallas as pl
from jax.experimental.pallas import tpu as pltpu
from jax.experimental.pallas import tpu_sc as plsc

_NC = 2
_NS = 16
_NW = _NC * _NS
_CH = 16
_U = 8
_K = 4
_P = 2


def _sc_add(x2, pos2, B, S, D):
    rows_per_w = S // _NW
    chunks = rows_per_w // _CH
    NJ = chunks * B
    VECS = (_CH * D) // 16
    minor_vecs = D // 16

    mesh = plsc.VectorSubcoreMesh(core_axis_name="c", subcore_axis_name="s")

    @functools.partial(
        pl.kernel,
        mesh=mesh,
        out_type=jax.ShapeDtypeStruct((B * S, D), jnp.float32),
        compiler_params=pltpu.CompilerParams(
            disable_bounds_checks=True,
            disable_semaphore_checks=True,
        ),
        scratch_types=[
            pltpu.VMEM((2, _CH, D), jnp.float32),
            pltpu.VMEM((_K, _CH, D), jnp.float32),
            pltpu.SemaphoreType.DMA((2,)),
            pltpu.SemaphoreType.DMA((_K,)),
            pltpu.SemaphoreType.DMA((_K,)),
        ],
    )
    def k(x_hbm, pos_hbm, out_hbm, pos_buf, x_buf, pos_sem, in_sem, out_sem):
        wid = lax.axis_index("s") * _NC + lax.axis_index("c")
        base_row = wid * rows_per_w

        def pos_slice(c):
            return pos_hbm.at[pl.ds(pl.multiple_of(base_row + c * _CH, _CH), _CH), :]

        def x_slice(hbm, c, b):
            row = pl.multiple_of(b * S + base_row + c * _CH, _CH)
            return hbm.at[pl.ds(row, _CH), :]

        def issue_pos(c, ps):
            pltpu.async_copy(pos_slice(c), pos_buf.at[ps], pos_sem.at[ps])

        def wait_pos(c, ps):
            pltpu.make_async_copy(pos_slice(c), pos_buf.at[ps],
                                  pos_sem.at[ps]).wait()

        def issue_in(c, b, slot):
            pltpu.async_copy(x_slice(x_hbm, c, b), x_buf.at[slot],
                             in_sem.at[slot])

        def wait_in(c, b, slot):
            pltpu.make_async_copy(x_slice(x_hbm, c, b), x_buf.at[slot],
                                  in_sem.at[slot]).wait()

        def issue_out(c, b, slot):
            pltpu.async_copy(x_buf.at[slot], x_slice(out_hbm, c, b),
                             out_sem.at[slot])

        def wait_out(c, b, slot):
            pltpu.make_async_copy(x_buf.at[slot], x_slice(out_hbm, c, b),
                                  out_sem.at[slot]).wait()

        issue_pos(0, 0)
        for j in range(_P):
            issue_in(j // B, j % B, j % _K)

        def job_body(j, carry):
            c = lax.div(j, B)
            b = lax.rem(j, B)
            slot = lax.rem(j, _K)
            ps = lax.rem(c, 2)

            @pl.when(b == 0)
            def _pos():
                @pl.when(c + 1 < chunks)
                def _():
                    issue_pos(c + 1, lax.rem(c + 1, 2))
                wait_pos(c, ps)

            nj = j + _P

            @pl.when(nj < NJ)
            def _prefetch():
                nslot = lax.rem(nj, _K)
                pj = nj - _K

                @pl.when(pj >= 0)
                def _recycle():
                    wait_out(lax.div(pj, B), lax.rem(pj, B), lax.rem(pj, _K))

                issue_in(lax.div(nj, B), lax.rem(nj, B), nslot)

            wait_in(c, b, slot)

            @plsc.parallel_loop(0, VECS, step=1, unroll=_U)
            def add_u(i):
                r = i // minor_vecs
                sl = pl.ds((i % minor_vecs) * 16, 16)
                x_buf[slot, r, sl] = x_buf[slot, r, sl] + pos_buf[ps, r, sl]

            issue_out(c, b, slot)
            return carry

        lax.fori_loop(0, NJ, job_body, 0)

        for j in range(max(NJ - _K, 0), NJ):
            wait_out(j // B, j % B, j % _K)

    return k(x2, pos2)


def kernel(x, pos_table):
    B, S, D = x.shape
    out2 = _sc_add(x.reshape(B * S, D), pos_table[:S], B, S, D)
    return out2.reshape(B, S, D)
